# Initial kernel scaffold; baseline (speedup 1.0000x reference)
import jax
import jax.numpy as jnp
from jax.experimental import pallas as pl


def kernel(x, embedding):
    raise NotImplementedError("write your pallas kernel here")


# speedup vs baseline: 4.8391x; 4.8391x over previous
import functools

import jax
import jax.numpy as jnp
from jax import lax
from jax.experimental import pallas as pl
from jax.experimental.pallas import tpu as pltpu
from jax.experimental.pallas import tpu_sc as plsc

NE = 8192
ED = 32
NB = 512
EC = 1024
BATCH = 8
HW = 64 * 64
NTOK = BATCH * HW
BETA = 0.25


def _argmin_body(x_ref, e_ref, idx_ref, loss_ref, en_ref):
    b = pl.program_id(0)
    nb = pl.program_id(1)
    first = (b == 0) & (nb == 0)

    @pl.when(first)
    def _():
        e = e_ref[...]
        en_ref[...] = jnp.sum(e * e, axis=1, keepdims=True)

    xb = x_ref[0]
    run_min = jnp.full((1, NB), jnp.inf, dtype=jnp.float32)
    run_idx = jnp.zeros((1, NB), dtype=jnp.int32)
    for c in range(NE // EC):
        e_chunk = e_ref[pl.ds(c * EC, EC), :]
        sim = lax.dot_general(e_chunk, xb, (((1,), (0,)), ((), ())),
                              preferred_element_type=jnp.float32)
        d = en_ref[pl.ds(c * EC, EC), :] - 2.0 * sim
        cmin = jnp.min(d, axis=0, keepdims=True)
        ii = lax.broadcasted_iota(jnp.int32, (EC, NB), 0)
        cidx = jnp.min(jnp.where(d == cmin, ii, jnp.int32(2**30)),
                       axis=0, keepdims=True) + c * EC
        better = cmin < run_min
        run_idx = jnp.where(better, cidx, run_idx)
        run_min = jnp.where(better, cmin, run_min)

    idx_ref[0] = run_idx
    part = jnp.sum(xb * xb) + jnp.sum(run_min)

    @pl.when(first)
    def _():
        loss_ref[...] = jnp.zeros_like(loss_ref)

    loss_ref[...] += part[None, None]

    @pl.when((b == BATCH - 1) & (nb == HW // NB - 1))
    def _():
        loss_ref[...] *= (1.0 + BETA) / (NTOK * ED)


def _compute_indices(xr, emb):
    grid = (BATCH, HW // NB)
    idx3, loss = pl.pallas_call(
        _argmin_body,
        grid=grid,
        in_specs=[
            pl.BlockSpec((1, ED, NB), lambda b, n: (b, 0, n)),
            pl.BlockSpec((NE, ED), lambda b, n: (0, 0)),
        ],
        out_specs=[
            pl.BlockSpec((1, 1, NB), lambda b, n: (b * (HW // NB) + n, 0, 0)),
            pl.BlockSpec((1, 1), lambda b, n: (0, 0)),
        ],
        out_shape=[
            jax.ShapeDtypeStruct((NTOK // NB, 1, NB), jnp.int32),
            jax.ShapeDtypeStruct((1, 1), jnp.float32),
        ],
        scratch_shapes=[pltpu.VMEM((NE, 1), jnp.float32)],
    )(xr, emb)
    return idx3.reshape(NTOK), loss.reshape(())


def _gather_channels(et, idx):
    mesh = plsc.VectorSubcoreMesh(core_axis_name="c", subcore_axis_name="s")

    @functools.partial(
        pl.kernel,
        mesh=mesh,
        out_type=jax.ShapeDtypeStruct((BATCH, ED, HW), jnp.float32),
        scratch_types=[
            pltpu.VMEM((NTOK,), jnp.int32),
            pltpu.VMEM((NE,), jnp.float32),
            pltpu.VMEM((BATCH, HW), jnp.float32),
        ],
        compiler_params=pltpu.CompilerParams(needs_layout_passes=False),
    )
    def k(et_hbm, idx_hbm, out_hbm, idx_v, row_v, out_v):
        wid = lax.axis_index("s") * 2 + lax.axis_index("c")
        pltpu.sync_copy(idx_hbm, idx_v)
        pltpu.sync_copy(et_hbm.at[wid], row_v)
        for b in range(BATCH):
            def body(i, carry):
                s = i * 16
                iv = idx_v[pl.ds(b * HW + s, 16)]
                out_v[b, pl.ds(s, 16)] = plsc.load_gather(row_v, [iv])
                return carry
            lax.fori_loop(0, HW // 16, body, 0)
            pltpu.sync_copy(out_v.at[b], out_hbm.at[b, wid])

    return k(et, idx)


def kernel(x, embedding):
    xr = x.reshape(BATCH, ED, HW)
    idx, loss = _compute_indices(xr, embedding)
    et = embedding.T
    xq = _gather_channels(et, idx)
    return xq.reshape(x.shape), loss


# speedup vs baseline: 4.9806x; 1.0292x over previous
import functools

import jax
import jax.numpy as jnp
from jax import lax
from jax.experimental import pallas as pl
from jax.experimental.pallas import tpu as pltpu
from jax.experimental.pallas import tpu_sc as plsc

NE = 8192
ED = 32
NB = 512
EC = 1024
BATCH = 8
HW = 64 * 64
NTOK = BATCH * HW
BETA = 0.25


def _argmin_body(x_ref, e_ref, idx_ref, loss_ref, ep_ref, en_ref):
    b = pl.program_id(0)
    nb = pl.program_id(1)
    first = (b == 0) & (nb == 0)

    @pl.when(first)
    def _():
        e = e_ref[...]
        ep_ref[...] = -2.0 * e
        en_ref[...] = jnp.sum(e * e, axis=1, keepdims=True)

    xb = x_ref[0]
    run_min = jnp.full((1, NB), jnp.inf, dtype=jnp.float32)
    run_idx = jnp.zeros((1, NB), dtype=jnp.float32)
    ii = lax.broadcasted_iota(jnp.int32, (EC, NB), 0).astype(jnp.float32)
    for c in range(NE // EC):
        e_chunk = ep_ref[pl.ds(c * EC, EC), :]
        sim = lax.dot_general(e_chunk, xb, (((1,), (0,)), ((), ())),
                              preferred_element_type=jnp.float32)
        d = en_ref[pl.ds(c * EC, EC), :] + sim
        cmin = jnp.min(d, axis=0, keepdims=True)
        cidx = jnp.min(jnp.where(d == cmin, ii, jnp.float32(3e38)),
                       axis=0, keepdims=True) + float(c * EC)
        better = cmin < run_min
        run_idx = jnp.where(better, cidx, run_idx)
        run_min = jnp.where(better, cmin, run_min)

    idx_ref[0] = run_idx.astype(jnp.int32)
    part = jnp.sum(xb * xb) + jnp.sum(run_min)

    @pl.when(first)
    def _():
        loss_ref[...] = jnp.zeros_like(loss_ref)

    loss_ref[...] += part[None, None]

    @pl.when((b == BATCH - 1) & (nb == HW // NB - 1))
    def _():
        loss_ref[...] *= (1.0 + BETA) / (NTOK * ED)


def _compute_indices(xr, emb):
    grid = (BATCH, HW // NB)
    idx3, loss = pl.pallas_call(
        _argmin_body,
        grid=grid,
        in_specs=[
            pl.BlockSpec((1, ED, NB), lambda b, n: (b, 0, n)),
            pl.BlockSpec((NE, ED), lambda b, n: (0, 0)),
        ],
        out_specs=[
            pl.BlockSpec((1, 1, NB), lambda b, n: (b * (HW // NB) + n, 0, 0)),
            pl.BlockSpec((1, 1), lambda b, n: (0, 0)),
        ],
        out_shape=[
            jax.ShapeDtypeStruct((NTOK // NB, 1, NB), jnp.int32),
            jax.ShapeDtypeStruct((1, 1), jnp.float32),
        ],
        scratch_shapes=[pltpu.VMEM((NE, ED), jnp.float32),
                        pltpu.VMEM((NE, 1), jnp.float32)],
    )(xr, emb)
    return idx3.reshape(NTOK), loss.reshape(())


def _gather_channels(et, idx):
    mesh = plsc.VectorSubcoreMesh(core_axis_name="c", subcore_axis_name="s")

    @functools.partial(
        pl.kernel,
        mesh=mesh,
        out_type=jax.ShapeDtypeStruct((BATCH, ED, HW), jnp.float32),
        scratch_types=[
            pltpu.VMEM((NTOK,), jnp.int32),
            pltpu.VMEM((NE,), jnp.float32),
            pltpu.VMEM((BATCH, HW), jnp.float32),
        ],
        compiler_params=pltpu.CompilerParams(needs_layout_passes=False),
    )
    def k(et_hbm, idx_hbm, out_hbm, idx_v, row_v, out_v):
        wid = lax.axis_index("s") * 2 + lax.axis_index("c")
        pltpu.sync_copy(idx_hbm, idx_v)
        pltpu.sync_copy(et_hbm.at[wid], row_v)
        for b in range(BATCH):
            def body(i, carry):
                s = i * 16
                iv = idx_v[pl.ds(b * HW + s, 16)]
                out_v[b, pl.ds(s, 16)] = plsc.load_gather(row_v, [iv])
                return carry
            lax.fori_loop(0, HW // 16, body, 0)
            pltpu.sync_copy(out_v.at[b], out_hbm.at[b, wid])

    return k(et, idx)


def kernel(x, embedding):
    xr = x.reshape(BATCH, ED, HW)
    idx, loss = _compute_indices(xr, embedding)
    et = embedding.T
    xq = _gather_channels(et, idx)
    return xq.reshape(x.shape), loss


# speedup vs baseline: 6.5489x; 1.3149x over previous
import functools

import jax
import jax.numpy as jnp
from jax import lax
from jax.experimental import pallas as pl
from jax.experimental.pallas import tpu as pltpu
from jax.experimental.pallas import tpu_sc as plsc

NE = 8192
ED = 32
NB = 512
EC = 1024
BATCH = 8
HW = 64 * 64
NTOK = BATCH * HW
BETA = 0.25


def _argmin_body(x_ref, e_ref, idx_ref, loss_ref, ep_ref, en_ref):
    b = pl.program_id(0)
    nb = pl.program_id(1)
    first = (b == 0) & (nb == 0)

    @pl.when(first)
    def _():
        e = e_ref[...]
        ep_ref[...] = -2.0 * e
        en_ref[...] = jnp.sum(e * e, axis=1, keepdims=True)

    xb = x_ref[0]
    run_min = jnp.full((1, NB), jnp.inf, dtype=jnp.float32)
    run_idx = jnp.zeros((1, NB), dtype=jnp.int32)
    for c in range(NE // EC):
        e_chunk = ep_ref[pl.ds(c * EC, EC), :]
        sim = lax.dot_general(e_chunk, xb, (((1,), (0,)), ((), ())),
                              preferred_element_type=jnp.float32)
        d = en_ref[pl.ds(c * EC, EC), :] + sim
        cmin = jnp.min(d, axis=0, keepdims=True)
        cidx = jnp.argmin(d, axis=0)[None, :] + c * EC
        better = cmin < run_min
        run_idx = jnp.where(better, cidx, run_idx)
        run_min = jnp.where(better, cmin, run_min)

    idx_ref[0] = run_idx
    part = jnp.sum(xb * xb) + jnp.sum(run_min)

    @pl.when(first)
    def _():
        loss_ref[...] = jnp.zeros_like(loss_ref)

    loss_ref[...] += part[None, None]

    @pl.when((b == BATCH - 1) & (nb == HW // NB - 1))
    def _():
        loss_ref[...] *= (1.0 + BETA) / (NTOK * ED)


def _compute_indices(xr, emb):
    grid = (BATCH, HW // NB)
    idx3, loss = pl.pallas_call(
        _argmin_body,
        grid=grid,
        in_specs=[
            pl.BlockSpec((1, ED, NB), lambda b, n: (b, 0, n)),
            pl.BlockSpec((NE, ED), lambda b, n: (0, 0)),
        ],
        out_specs=[
            pl.BlockSpec((1, 1, NB), lambda b, n: (b * (HW // NB) + n, 0, 0)),
            pl.BlockSpec((1, 1), lambda b, n: (0, 0)),
        ],
        out_shape=[
            jax.ShapeDtypeStruct((NTOK // NB, 1, NB), jnp.int32),
            jax.ShapeDtypeStruct((1, 1), jnp.float32),
        ],
        scratch_shapes=[pltpu.VMEM((NE, ED), jnp.float32),
                        pltpu.VMEM((NE, 1), jnp.float32)],
    )(xr, emb)
    return idx3.reshape(NTOK), loss.reshape(())


def _gather_channels(et, idx):
    mesh = plsc.VectorSubcoreMesh(core_axis_name="c", subcore_axis_name="s")

    @functools.partial(
        pl.kernel,
        mesh=mesh,
        out_type=jax.ShapeDtypeStruct((BATCH, ED, HW), jnp.float32),
        scratch_types=[
            pltpu.VMEM((NTOK,), jnp.int32),
            pltpu.VMEM((NE,), jnp.float32),
            pltpu.VMEM((BATCH, HW), jnp.float32),
        ],
        compiler_params=pltpu.CompilerParams(needs_layout_passes=False),
    )
    def k(et_hbm, idx_hbm, out_hbm, idx_v, row_v, out_v):
        wid = lax.axis_index("s") * 2 + lax.axis_index("c")
        pltpu.sync_copy(idx_hbm, idx_v)
        pltpu.sync_copy(et_hbm.at[wid], row_v)
        for b in range(BATCH):
            def body(i, carry):
                s = i * 16
                iv = idx_v[pl.ds(b * HW + s, 16)]
                out_v[b, pl.ds(s, 16)] = plsc.load_gather(row_v, [iv])
                return carry
            lax.fori_loop(0, HW // 16, body, 0)
            pltpu.sync_copy(out_v.at[b], out_hbm.at[b, wid])

    return k(et, idx)


def kernel(x, embedding):
    xr = x.reshape(BATCH, ED, HW)
    idx, loss = _compute_indices(xr, embedding)
    et = embedding.T
    xq = _gather_channels(et, idx)
    return xq.reshape(x.shape), loss


# speedup vs baseline: 7.5955x; 1.1598x over previous
import functools

import jax
import jax.numpy as jnp
from jax import lax
from jax.experimental import pallas as pl
from jax.experimental.pallas import tpu as pltpu
from jax.experimental.pallas import tpu_sc as plsc

NE = 8192
ED = 32
NB = 512
EC = 1024
BATCH = 8
HW = 64 * 64
NTOK = BATCH * HW
BETA = 0.25


def _argmin_body(x_ref, e_ref, idx_ref, ep_ref, en_ref, d_ref):
    first = (pl.program_id(0) == 0) & (pl.program_id(1) == 0)

    @pl.when(first)
    def _():
        e = e_ref[...]
        ep_ref[...] = -2.0 * e
        en_ref[...] = jnp.sum(e * e, axis=1, keepdims=True)

    xb = x_ref[0]
    for c in range(NE // EC):
        e_chunk = ep_ref[pl.ds(c * EC, EC), :]
        sim = lax.dot_general(e_chunk, xb, (((1,), (0,)), ((), ())),
                              preferred_element_type=jnp.float32)
        d_ref[pl.ds(c * EC, EC), :] = en_ref[pl.ds(c * EC, EC), :] + sim

    idx_ref[0] = jnp.argmin(d_ref[...], axis=0)[None, :]


def _compute_indices(xr, emb):
    grid = (BATCH, HW // NB)
    idx3 = pl.pallas_call(
        _argmin_body,
        grid=grid,
        in_specs=[
            pl.BlockSpec((1, ED, NB), lambda b, n: (b, 0, n)),
            pl.BlockSpec((NE, ED), lambda b, n: (0, 0)),
        ],
        out_specs=pl.BlockSpec((1, 1, NB), lambda b, n: (b * (HW // NB) + n, 0, 0)),
        out_shape=jax.ShapeDtypeStruct((NTOK // NB, 1, NB), jnp.int32),
        scratch_shapes=[pltpu.VMEM((NE, ED), jnp.float32),
                        pltpu.VMEM((NE, 1), jnp.float32),
                        pltpu.VMEM((NE, NB), jnp.float32)],
    )(xr, emb)
    return idx3.reshape(NTOK)


def _gather_channels(et, xr, idx):
    mesh = plsc.VectorSubcoreMesh(core_axis_name="c", subcore_axis_name="s")

    @functools.partial(
        pl.kernel,
        mesh=mesh,
        out_type=[jax.ShapeDtypeStruct((BATCH, ED, HW), jnp.float32),
                  jax.ShapeDtypeStruct((ED, 16), jnp.float32)],
        scratch_types=[
            pltpu.VMEM((NTOK,), jnp.int32),
            pltpu.VMEM((NE,), jnp.float32),
            pltpu.VMEM((BATCH, HW), jnp.float32),
            pltpu.VMEM((HW,), jnp.float32),
            pltpu.VMEM((16,), jnp.float32),
        ],
        compiler_params=pltpu.CompilerParams(needs_layout_passes=False),
    )
    def k(et_hbm, xr_hbm, idx_hbm, out_hbm, lossp_hbm,
          idx_v, row_v, out_v, xr_v, acc_v):
        wid = lax.axis_index("s") * 2 + lax.axis_index("c")
        pltpu.sync_copy(idx_hbm, idx_v)
        pltpu.sync_copy(et_hbm.at[wid], row_v)
        acc = jnp.zeros((16,), jnp.float32)
        for b in range(BATCH):
            pltpu.sync_copy(xr_hbm.at[b, wid], xr_v)

            def body(i, a):
                s = i * 16
                g = plsc.load_gather(row_v, [idx_v[pl.ds(b * HW + s, 16)]])
                out_v[b, pl.ds(s, 16)] = g
                dv = g - xr_v[pl.ds(s, 16)]
                return a + dv * dv

            acc = lax.fori_loop(0, HW // 16, body, acc)
            pltpu.sync_copy(out_v.at[b], out_hbm.at[b, wid])
        acc_v[...] = acc
        pltpu.sync_copy(acc_v, lossp_hbm.at[wid])

    return k(et, xr, idx)


def _loss_body(lp_ref, loss_ref):
    loss_ref[...] = (jnp.sum(lp_ref[...]) *
                     ((1.0 + BETA) / (NTOK * ED)))[None, None]


def _finish_loss(lossp):
    loss = pl.pallas_call(
        _loss_body,
        out_shape=jax.ShapeDtypeStruct((1, 1), jnp.float32),
    )(lossp)
    return loss.reshape(())


def kernel(x, embedding):
    xr = x.reshape(BATCH, ED, HW)
    idx = _compute_indices(xr, embedding)
    et = embedding.T
    xq, lossp = _gather_channels(et, xr, idx)
    loss = _finish_loss(lossp)
    return xq.reshape(x.shape), loss


# speedup vs baseline: 7.8948x; 1.0394x over previous
import functools

import jax
import jax.numpy as jnp
from jax import lax
from jax.experimental import pallas as pl
from jax.experimental.pallas import tpu as pltpu
from jax.experimental.pallas import tpu_sc as plsc

NE = 8192
ED = 32
NB = 512
EC = 1024
BATCH = 8
HW = 64 * 64
NTOK = BATCH * HW
BETA = 0.25


def _argmin_body(x_ref, e_ref, idx_ref, ep_ref, en_ref, da_ref, db_ref):
    first = (pl.program_id(0) == 0) & (pl.program_id(1) == 0)

    @pl.when(first)
    def _():
        e = e_ref[...]
        ep_ref[...] = -2.0 * e
        en_ref[...] = jnp.sum(e * e, axis=1, keepdims=True)

    xb = x_ref[0]
    for d_ref, lo in ((da_ref, 0), (db_ref, NB)):
        xh = xb[:, lo:lo + NB]
        for c in range(NE // EC):
            e_chunk = ep_ref[pl.ds(c * EC, EC), :]
            sim = lax.dot_general(e_chunk, xh, (((1,), (0,)), ((), ())),
                                  preferred_element_type=jnp.float32)
            d_ref[pl.ds(c * EC, EC), :] = en_ref[pl.ds(c * EC, EC), :] + sim

    ia = jnp.argmin(da_ref[...], axis=0)
    ib = jnp.argmin(db_ref[...], axis=0)
    idx_ref[0] = jnp.concatenate([ia, ib])[None, :]


def _compute_indices(xr, emb):
    grid = (BATCH, HW // (2 * NB))
    idx3 = pl.pallas_call(
        _argmin_body,
        grid=grid,
        in_specs=[
            pl.BlockSpec((1, ED, 2 * NB), lambda b, n: (b, 0, n)),
            pl.BlockSpec((NE, ED), lambda b, n: (0, 0)),
        ],
        out_specs=pl.BlockSpec((1, 1, 2 * NB),
                               lambda b, n: (b * (HW // (2 * NB)) + n, 0, 0)),
        out_shape=jax.ShapeDtypeStruct((NTOK // (2 * NB), 1, 2 * NB), jnp.int32),
        scratch_shapes=[pltpu.VMEM((NE, ED), jnp.float32),
                        pltpu.VMEM((NE, 1), jnp.float32),
                        pltpu.VMEM((NE, NB), jnp.float32),
                        pltpu.VMEM((NE, NB), jnp.float32)],
    )(xr, emb)
    return idx3.reshape(NTOK)


def _gather_channels(et, xr, idx):
    mesh = plsc.VectorSubcoreMesh(core_axis_name="c", subcore_axis_name="s")

    @functools.partial(
        pl.kernel,
        mesh=mesh,
        out_type=[jax.ShapeDtypeStruct((BATCH, ED, HW), jnp.float32),
                  jax.ShapeDtypeStruct((ED, 16), jnp.float32)],
        scratch_types=[
            pltpu.VMEM((NTOK,), jnp.int32),
            pltpu.VMEM((NE,), jnp.float32),
            pltpu.VMEM((BATCH, HW), jnp.float32),
            pltpu.VMEM((HW,), jnp.float32),
            pltpu.VMEM((16,), jnp.float32),
        ],
        compiler_params=pltpu.CompilerParams(needs_layout_passes=False),
    )
    def k(et_hbm, xr_hbm, idx_hbm, out_hbm, lossp_hbm,
          idx_v, row_v, out_v, xr_v, acc_v):
        wid = lax.axis_index("s") * 2 + lax.axis_index("c")
        pltpu.sync_copy(idx_hbm, idx_v)
        pltpu.sync_copy(et_hbm.at[wid], row_v)
        acc = jnp.zeros((16,), jnp.float32)
        for b in range(BATCH):
            pltpu.sync_copy(xr_hbm.at[b, wid], xr_v)

            def body(i, a):
                s = i * 16
                g = plsc.load_gather(row_v, [idx_v[pl.ds(b * HW + s, 16)]])
                out_v[b, pl.ds(s, 16)] = g
                dv = g - xr_v[pl.ds(s, 16)]
                return a + dv * dv

            acc = lax.fori_loop(0, HW // 16, body, acc)
            pltpu.sync_copy(out_v.at[b], out_hbm.at[b, wid])
        acc_v[...] = acc
        pltpu.sync_copy(acc_v, lossp_hbm.at[wid])

    return k(et, xr, idx)


def _loss_body(lp_ref, loss_ref):
    loss_ref[...] = (jnp.sum(lp_ref[...]) *
                     ((1.0 + BETA) / (NTOK * ED)))[None, None]


def _finish_loss(lossp):
    loss = pl.pallas_call(
        _loss_body,
        out_shape=jax.ShapeDtypeStruct((1, 1), jnp.float32),
    )(lossp)
    return loss.reshape(())


def kernel(x, embedding):
    xr = x.reshape(BATCH, ED, HW)
    idx = _compute_indices(xr, embedding)
    et = embedding.T
    xq, lossp = _gather_channels(et, xr, idx)
    loss = _finish_loss(lossp)
    return xq.reshape(x.shape), loss


# speedup vs baseline: 8.0868x; 1.0243x over previous
import functools

import jax
import jax.numpy as jnp
from jax import lax
from jax.experimental import pallas as pl
from jax.experimental.pallas import tpu as pltpu
from jax.experimental.pallas import tpu_sc as plsc

NE = 8192
ED = 32
NB = 512
EC = 1024
BATCH = 8
HW = 64 * 64
NTOK = BATCH * HW
BETA = 0.25


def _argmin_body(x_ref, e_ref, idx_ref, ep_ref, en_ref, da_ref, db_ref):
    first = (pl.program_id(0) == 0) & (pl.program_id(1) == 0)

    @pl.when(first)
    def _():
        e = e_ref[...]
        ep_ref[...] = -2.0 * e
        en_ref[...] = jnp.sum(e * e, axis=1, keepdims=True)

    xb = x_ref[0]
    for d_ref, lo in ((da_ref, 0), (db_ref, NB)):
        xh = xb[:, lo:lo + NB]
        for c in range(NE // EC):
            e_chunk = ep_ref[pl.ds(c * EC, EC), :]
            sim = lax.dot_general(e_chunk, xh, (((1,), (0,)), ((), ())),
                                  preferred_element_type=jnp.float32)
            d_ref[pl.ds(c * EC, EC), :] = en_ref[pl.ds(c * EC, EC), :] + sim

    ia = jnp.argmin(da_ref[...], axis=0)
    ib = jnp.argmin(db_ref[...], axis=0)
    idx_ref[0] = jnp.concatenate([ia, ib])[None, :]


def _compute_indices(xr, emb):
    grid = (BATCH, HW // (2 * NB))
    idx3 = pl.pallas_call(
        _argmin_body,
        grid=grid,
        in_specs=[
            pl.BlockSpec((1, ED, 2 * NB), lambda b, n: (b, 0, n)),
            pl.BlockSpec((NE, ED), lambda b, n: (0, 0)),
        ],
        out_specs=pl.BlockSpec((1, 1, 2 * NB),
                               lambda b, n: (b * (HW // (2 * NB)) + n, 0, 0)),
        out_shape=jax.ShapeDtypeStruct((NTOK // (2 * NB), 1, 2 * NB), jnp.int32),
        scratch_shapes=[pltpu.VMEM((NE, ED), jnp.float32),
                        pltpu.VMEM((NE, 1), jnp.float32),
                        pltpu.VMEM((NE, NB), jnp.float32),
                        pltpu.VMEM((NE, NB), jnp.float32)],
    )(xr, emb)
    return idx3.reshape(NTOK)


def _gather_channels(et, xr, idx):
    mesh = plsc.VectorSubcoreMesh(core_axis_name="c", subcore_axis_name="s")

    @functools.partial(
        pl.kernel,
        mesh=mesh,
        out_type=[jax.ShapeDtypeStruct((BATCH, ED, HW), jnp.float32),
                  jax.ShapeDtypeStruct((ED, 16), jnp.float32)],
        scratch_types=[
            pltpu.VMEM((NTOK,), jnp.int32),
            pltpu.VMEM((NE,), jnp.float32),
            pltpu.VMEM((BATCH, HW), jnp.float32),
            pltpu.VMEM((BATCH, HW), jnp.float32),
            pltpu.VMEM((16,), jnp.float32),
            pltpu.SemaphoreType.DMA,
            pltpu.SemaphoreType.DMA,
            pltpu.SemaphoreType.DMA,
        ],
        compiler_params=pltpu.CompilerParams(needs_layout_passes=False),
    )
    def k(et_hbm, xr_hbm, idx_hbm, out_hbm, lossp_hbm,
          idx_v, row_v, out_v, xr_v, acc_v, sem_i, sem_r, sem_x):
        wid = lax.axis_index("s") * 2 + lax.axis_index("c")
        cp_i = pltpu.async_copy(idx_hbm, idx_v, sem_i)
        cp_r = pltpu.async_copy(et_hbm.at[wid], row_v, sem_r)
        cp_x = pltpu.async_copy(xr_hbm.at[:, wid], xr_v, sem_x)
        cp_i.wait()
        cp_r.wait()
        cp_x.wait()
        acc = jnp.zeros((16,), jnp.float32)
        for b in range(BATCH):
            def body(i, a):
                s = i * 16
                g = plsc.load_gather(row_v, [idx_v[pl.ds(b * HW + s, 16)]])
                out_v[b, pl.ds(s, 16)] = g
                dv = g - xr_v[b, pl.ds(s, 16)]
                return a + dv * dv

            acc = lax.fori_loop(0, HW // 16, body, acc)
        acc_v[...] = acc
        pltpu.sync_copy(out_v, out_hbm.at[:, wid])
        pltpu.sync_copy(acc_v, lossp_hbm.at[wid])

    return k(et, xr, idx)


def _loss_body(lp_ref, loss_ref):
    loss_ref[...] = (jnp.sum(lp_ref[...]) *
                     ((1.0 + BETA) / (NTOK * ED)))[None, None]


def _finish_loss(lossp):
    loss = pl.pallas_call(
        _loss_body,
        out_shape=jax.ShapeDtypeStruct((1, 1), jnp.float32),
    )(lossp)
    return loss.reshape(())


def kernel(x, embedding):
    xr = x.reshape(BATCH, ED, HW)
    idx = _compute_indices(xr, embedding)
    et = embedding.T
    xq, lossp = _gather_channels(et, xr, idx)
    loss = _finish_loss(lossp)
    return xq.reshape(x.shape), loss
